# baseline (speedup 1.0000x reference)
import functools

import jax
import jax.numpy as jnp
from jax import lax
from jax.experimental import pallas as pl
from jax.experimental.pallas import tpu as pltpu
from jax.experimental.pallas import tpu_sc as plsc

N = 10000
KN = 32
K = 15
KP = 16
CIN = 128
C1 = 64
CB = 32
COUT = 128
A = 16
H = 256

NC = 2
NS = 16
NW = NC * NS
PER_W = (N * KN) // NW
CHUNK = 400
N_CHUNKS = PER_W // CHUNK

BLK = 400
GRID = N // BLK
SB = 8
NSB = BLK // SB
SROW = SB * KP
SCOL = SB * KN


def _sc_gather_feat_pts(nbr_hbm, feat_hbm, px_hbm, py_hbm, pz_hbm,
                        nf_out, nx_out, ny_out, nz_out,
                        idx_v, rows_v, cx_v, cy_v, cz_v, sem, semc):
    wid = lax.axis_index("s") * NC + lax.axis_index("c")
    base = wid * PER_W

    def body(i, carry):
        off = base + i * CHUNK
        pltpu.sync_copy(nbr_hbm.at[pl.ds(off, CHUNK)], idx_v)
        cp_f = pltpu.async_copy(feat_hbm.at[idx_v], rows_v, sem)
        cp_x = pltpu.async_copy(px_hbm.at[idx_v], cx_v, semc)
        cp_y = pltpu.async_copy(py_hbm.at[idx_v], cy_v, semc)
        cp_z = pltpu.async_copy(pz_hbm.at[idx_v], cz_v, semc)
        cp_f.wait()
        pltpu.sync_copy(rows_v, nf_out.at[pl.ds(off, CHUNK)])
        cp_x.wait()
        cp_y.wait()
        cp_z.wait()
        pltpu.sync_copy(cx_v, nx_out.at[pl.ds(off, CHUNK)])
        pltpu.sync_copy(cy_v, ny_out.at[pl.ds(off, CHUNK)])
        pltpu.sync_copy(cz_v, nz_out.at[pl.ds(off, CHUNK)])
        return carry

    lax.fori_loop(0, N_CHUNKS, body, 0)


def _sc_gather_y(nbr_hbm, y_hbm, ny_out, idx_v, rows_v, sem):
    wid = lax.axis_index("s") * NC + lax.axis_index("c")
    base = wid * PER_W

    def body(i, carry):
        off = base + i * CHUNK
        pltpu.sync_copy(nbr_hbm.at[pl.ds(off, CHUNK)], idx_v)
        pltpu.async_copy(y_hbm.at[idx_v], rows_v, sem).wait()
        pltpu.sync_copy(rows_v, ny_out.at[pl.ds(off, CHUNK)])
        return carry

    lax.fori_loop(0, N_CHUNKS, body, 0)


def _leaky(v):
    return jnp.where(v >= 0, v, 0.1 * v)


def _influence(nx, ny, nz, qx, qy, qz, kp):
    dx = (nx - qx)[:, None, :] - kp[0][None, :, :]
    dy = (ny - qy)[:, None, :] - kp[1][None, :, :]
    dz = (nz - qz)[:, None, :] - kp[2][None, :, :]
    d2 = dx * dx + dy * dy + dz * dz
    dist = jnp.sqrt(d2 + 1e-12)
    return jnp.maximum(0.0, 1.0 - dist)


def _weighted(infl, rows, mask, cdim):
    tiles = infl.astype(jnp.bfloat16).reshape(NSB, SROW, KN)
    tiles = jnp.concatenate([tiles] * SB, axis=-1)
    s_all = tiles * mask[None]
    rows_sb = rows.reshape(NSB, SCOL, cdim)
    outs = [jnp.dot(s_all[j], rows_sb[j], preferred_element_type=jnp.float32)
            for j in range(NSB)]
    wcat = jnp.concatenate(outs, axis=0)
    return wcat.reshape(BLK, KP, cdim)


def _tc1_body(nf_ref, nx_ref, ny_ref, nz_ref, qx_ref, qy_ref, qz_ref,
              kp_ref, mask_ref, ws_ref, wra_ref, x_ref, y1_ref):
    infl = _influence(nx_ref[...], ny_ref[...], nz_ref[...],
                      qx_ref[...], qy_ref[...], qz_ref[...], kp_ref[...])
    w = _weighted(infl, nf_ref[...], mask_ref[...], CIN)
    wb = w.astype(jnp.bfloat16)
    acc = jnp.zeros((BLK, C1), dtype=jnp.float32)
    for p in range(K):
        acc = acc + jnp.dot(wb[:, p, :], ws_ref[p],
                            preferred_element_type=jnp.float32)
    x = _leaky(acc)
    x_ref[...] = x
    y1_ref[...] = _leaky(jnp.dot(x, wra_ref[...],
                                 preferred_element_type=jnp.float32)
                         ).astype(jnp.bfloat16)


def _tc2_body(nyr_ref, nx_ref, ny_ref, nz_ref, qx_ref, qy_ref, qz_ref,
              kp_ref, mask_ref, x_ref, wrk_ref, wrb_ref,
              wsc_ref, ba_ref, wh1_ref, bh1_ref, wh2_ref, bh2_ref,
              wq_ref, bq_ref, qout_ref, acc_ref):
    i = pl.program_id(0)
    infl = _influence(nx_ref[...], ny_ref[...], nz_ref[...],
                      qx_ref[...], qy_ref[...], qz_ref[...], kp_ref[...])
    w = _weighted(infl, nyr_ref[...], mask_ref[...], CB)
    wb = w.astype(jnp.bfloat16)
    yacc = jnp.zeros((BLK, CB), dtype=jnp.float32)
    for p in range(K):
        yacc = yacc + jnp.dot(wb[:, p, :], wrk_ref[p],
                              preferred_element_type=jnp.float32)
    y = _leaky(yacc)
    y = jnp.dot(y, wrb_ref[...], preferred_element_type=jnp.float32)
    x = x_ref[...]
    x2 = _leaky(y + jnp.dot(x, wsc_ref[...],
                            preferred_element_type=jnp.float32))
    partial = jnp.sum(x2, axis=0, keepdims=True)

    @pl.when(i == 0)
    def _():
        acc_ref[...] = partial

    @pl.when(i > 0)
    def _():
        acc_ref[...] = acc_ref[...] + partial

    @pl.when(i == GRID - 1)
    def _():
        g = acc_ref[...] * (1.0 / N)
        h = jnp.concatenate([g, ba_ref[...]], axis=1)
        h = jnp.maximum(0.0, jnp.dot(h, wh1_ref[...],
                                     preferred_element_type=jnp.float32)
                        + bh1_ref[...])
        h = jnp.maximum(0.0, jnp.dot(h, wh2_ref[...],
                                     preferred_element_type=jnp.float32)
                        + bh2_ref[...])
        qout_ref[...] = jnp.dot(h, wq_ref[...],
                                preferred_element_type=jnp.float32) \
            + bq_ref[...]


def _make_sc_gather_feat_pts():
    mesh = plsc.VectorSubcoreMesh(core_axis_name="c", subcore_axis_name="s",
                                  num_cores=NC, num_subcores=NS)
    return pl.kernel(
        _sc_gather_feat_pts,
        out_type=(
            jax.ShapeDtypeStruct((N * KN, CIN), jnp.bfloat16),
            jax.ShapeDtypeStruct((N * KN,), jnp.float32),
            jax.ShapeDtypeStruct((N * KN,), jnp.float32),
            jax.ShapeDtypeStruct((N * KN,), jnp.float32),
        ),
        mesh=mesh,
        compiler_params=pltpu.CompilerParams(use_tc_tiling_on_sc=False),
        scratch_types=[
            pltpu.VMEM((CHUNK,), jnp.int32),
            pltpu.VMEM((CHUNK, CIN), jnp.bfloat16),
            pltpu.VMEM((CHUNK,), jnp.float32),
            pltpu.VMEM((CHUNK,), jnp.float32),
            pltpu.VMEM((CHUNK,), jnp.float32),
            pltpu.SemaphoreType.DMA,
            pltpu.SemaphoreType.DMA,
        ],
    )


def _make_sc_gather_y():
    mesh = plsc.VectorSubcoreMesh(core_axis_name="c", subcore_axis_name="s",
                                  num_cores=NC, num_subcores=NS)
    return pl.kernel(
        _sc_gather_y,
        out_type=jax.ShapeDtypeStruct((N * KN, CB), jnp.bfloat16),
        mesh=mesh,
        compiler_params=pltpu.CompilerParams(use_tc_tiling_on_sc=False),
        scratch_types=[
            pltpu.VMEM((CHUNK,), jnp.int32),
            pltpu.VMEM((CHUNK, CB), jnp.bfloat16),
            pltpu.SemaphoreType.DMA,
        ],
    )


def _plane_specs():
    return [
        pl.BlockSpec((BLK, KN), lambda i: (i, 0)),
        pl.BlockSpec((BLK, KN), lambda i: (i, 0)),
        pl.BlockSpec((BLK, KN), lambda i: (i, 0)),
        pl.BlockSpec((BLK, 1), lambda i: (i, 0)),
        pl.BlockSpec((BLK, 1), lambda i: (i, 0)),
        pl.BlockSpec((BLK, 1), lambda i: (i, 0)),
        pl.BlockSpec((3, KP, 1), lambda i: (0, 0, 0)),
        pl.BlockSpec((SROW, SCOL), lambda i: (0, 0)),
    ]


def _make_tc1():
    return pl.pallas_call(
        _tc1_body,
        grid=(GRID,),
        in_specs=[pl.BlockSpec((BLK * KN, CIN), lambda i: (i, 0))]
        + _plane_specs()
        + [
            pl.BlockSpec((K, CIN, C1), lambda i: (0, 0, 0)),
            pl.BlockSpec((C1, CB), lambda i: (0, 0)),
        ],
        out_specs=[
            pl.BlockSpec((BLK, C1), lambda i: (i, 0)),
            pl.BlockSpec((BLK, CB), lambda i: (i, 0)),
        ],
        out_shape=[
            jax.ShapeDtypeStruct((N, C1), jnp.float32),
            jax.ShapeDtypeStruct((N, CB), jnp.bfloat16),
        ],
    )


def _make_tc2():
    return pl.pallas_call(
        _tc2_body,
        grid=(GRID,),
        in_specs=[pl.BlockSpec((BLK * KN, CB), lambda i: (i, 0))]
        + _plane_specs()
        + [
            pl.BlockSpec((BLK, C1), lambda i: (i, 0)),
            pl.BlockSpec((K, CB, CB), lambda i: (0, 0, 0)),
            pl.BlockSpec((CB, COUT), lambda i: (0, 0)),
            pl.BlockSpec((C1, COUT), lambda i: (0, 0)),
            pl.BlockSpec((1, A), lambda i: (0, 0)),
            pl.BlockSpec((COUT + A, H), lambda i: (0, 0)),
            pl.BlockSpec((1, H), lambda i: (0, 0)),
            pl.BlockSpec((H, H), lambda i: (0, 0)),
            pl.BlockSpec((1, H), lambda i: (0, 0)),
            pl.BlockSpec((H, 1), lambda i: (0, 0)),
            pl.BlockSpec((1, 1), lambda i: (0, 0)),
        ],
        out_specs=pl.BlockSpec((1, 1), lambda i: (0, 0)),
        out_shape=jax.ShapeDtypeStruct((1, 1), jnp.float32),
        scratch_shapes=[pltpu.VMEM((1, COUT), jnp.float32)],
    )


def _block_diag_mask():
    r = jnp.arange(SROW)[:, None] // KP
    c = jnp.arange(SCOL)[None, :] // KN
    return (r == c).astype(jnp.bfloat16)


def kernel(features, points, neighbors, batch_action, kernel_points,
           W_simple, W_ra, W_rk, W_rb, W_sc, Wh1, bh1, Wh2, bh2, Wq, bq):
    nbr = neighbors.reshape(-1).astype(jnp.int32)
    pts = points.astype(jnp.float32)
    px, py, pz = pts[:, 0], pts[:, 1], pts[:, 2]
    kp = jnp.pad(kernel_points.astype(jnp.float32).T, ((0, 0), (0, 1)),
                 constant_values=1e3).reshape(3, KP, 1)
    mask = _block_diag_mask()

    nf_flat, nx_f, ny_f, nz_f = _make_sc_gather_feat_pts()(
        nbr, features.astype(jnp.bfloat16), px, py, pz)
    nx = nx_f.reshape(N, KN)
    nyc = ny_f.reshape(N, KN)
    nz = nz_f.reshape(N, KN)
    qx = px.reshape(N, 1)
    qy = py.reshape(N, 1)
    qz = pz.reshape(N, 1)

    x, y1 = _make_tc1()(nf_flat, nx, nyc, nz, qx, qy, qz, kp, mask,
                        W_simple.astype(jnp.bfloat16), W_ra)

    nyr_flat = _make_sc_gather_y()(nbr, y1)

    q = _make_tc2()(nyr_flat, nx, nyc, nz, qx, qy, qz, kp, mask, x,
                    W_rk.astype(jnp.bfloat16), W_rb, W_sc,
                    batch_action, Wh1, bh1.reshape(1, H),
                    Wh2, bh2.reshape(1, H), Wq, bq.reshape(1, 1))
    return q


# speedup vs baseline: 1.4555x; 1.4555x over previous
import functools

import jax
import jax.numpy as jnp
from jax import lax
from jax.experimental import pallas as pl
from jax.experimental.pallas import tpu as pltpu
from jax.experimental.pallas import tpu_sc as plsc

N = 10000
KN = 32
K = 15
KP = 16
CIN = 128
C1 = 64
CB = 32
COUT = 128
A = 16
H = 256

NC = 2
NS = 16
NW = NC * NS
PER_W = (N * KN) // NW
CHUNK = 400
N_CHUNKS = PER_W // CHUNK

BLK = 400
GRID = N // BLK
SB = 8
NSB = BLK // SB
SROW = SB * KP
SCOL = SB * KN


def _sc_gather_feat_pts(nbr_hbm, feat_hbm, px_hbm, py_hbm, pz_hbm,
                        nf_out, nx_out, ny_out, nz_out,
                        idx_a, idx_b, rows_a, rows_b, cx_a, cx_b,
                        cy_a, cy_b, cz_a, cz_b, gs_a, gs_b, ws_a, ws_b):
    wid = lax.axis_index("s") * NC + lax.axis_index("c")
    base = wid * PER_W

    def fire(off, idx_v, rows_v, cx_v, cy_v, cz_v, gs):
        pltpu.sync_copy(nbr_hbm.at[pl.ds(off, CHUNK)], idx_v)
        return (pltpu.async_copy(feat_hbm.at[idx_v], rows_v, gs),
                pltpu.async_copy(px_hbm.at[idx_v], cx_v, gs),
                pltpu.async_copy(py_hbm.at[idx_v], cy_v, gs),
                pltpu.async_copy(pz_hbm.at[idx_v], cz_v, gs))

    def writeback(off, rows_v, cx_v, cy_v, cz_v, ws):
        return (pltpu.async_copy(rows_v, nf_out.at[pl.ds(off, CHUNK)], ws),
                pltpu.async_copy(cx_v, nx_out.at[pl.ds(off, CHUNK)], ws),
                pltpu.async_copy(cy_v, ny_out.at[pl.ds(off, CHUNK)], ws),
                pltpu.async_copy(cz_v, nz_out.at[pl.ds(off, CHUNK)], ws))

    def body(j, carry):
        off0 = base + (2 * j) * CHUNK
        off1 = off0 + CHUNK
        cps_a = fire(off0, idx_a, rows_a, cx_a, cy_a, cz_a, gs_a)
        cps_b = fire(off1, idx_b, rows_b, cx_b, cy_b, cz_b, gs_b)
        for c in cps_a:
            c.wait()
        wb_a = writeback(off0, rows_a, cx_a, cy_a, cz_a, ws_a)
        for c in cps_b:
            c.wait()
        wb_b = writeback(off1, rows_b, cx_b, cy_b, cz_b, ws_b)
        for c in wb_a:
            c.wait()
        for c in wb_b:
            c.wait()
        return carry

    lax.fori_loop(0, N_CHUNKS // 2, body, 0)


def _sc_gather_y(nbr_hbm, y_hbm, ny_out, idx_a, idx_b, rows_a, rows_b,
                 gs_a, gs_b, ws_a, ws_b):
    wid = lax.axis_index("s") * NC + lax.axis_index("c")
    base = wid * PER_W

    def body(j, carry):
        off0 = base + (2 * j) * CHUNK
        off1 = off0 + CHUNK
        pltpu.sync_copy(nbr_hbm.at[pl.ds(off0, CHUNK)], idx_a)
        cp_a = pltpu.async_copy(y_hbm.at[idx_a], rows_a, gs_a)
        pltpu.sync_copy(nbr_hbm.at[pl.ds(off1, CHUNK)], idx_b)
        cp_b = pltpu.async_copy(y_hbm.at[idx_b], rows_b, gs_b)
        cp_a.wait()
        wb_a = pltpu.async_copy(rows_a, ny_out.at[pl.ds(off0, CHUNK)], ws_a)
        cp_b.wait()
        wb_b = pltpu.async_copy(rows_b, ny_out.at[pl.ds(off1, CHUNK)], ws_b)
        wb_a.wait()
        wb_b.wait()
        return carry

    lax.fori_loop(0, N_CHUNKS // 2, body, 0)


def _leaky(v):
    return jnp.where(v >= 0, v, 0.1 * v)


def _influence(nx, ny, nz, qx, qy, qz, kp):
    dx = (nx - qx)[:, None, :] - kp[0][None, :, :]
    dy = (ny - qy)[:, None, :] - kp[1][None, :, :]
    dz = (nz - qz)[:, None, :] - kp[2][None, :, :]
    d2 = dx * dx + dy * dy + dz * dz
    dist = jnp.sqrt(d2 + 1e-12)
    return jnp.maximum(0.0, 1.0 - dist)


def _weighted(infl, rows, mask, cdim):
    tiles = infl.reshape(NSB, SROW, KN)
    tiles = jnp.concatenate([tiles] * SB, axis=-1)
    s_all = tiles * mask[None]
    rows_sb = rows.reshape(NSB, SCOL, cdim)
    outs = [jnp.dot(s_all[j], rows_sb[j], preferred_element_type=jnp.float32)
            for j in range(NSB)]
    wcat = jnp.concatenate(outs, axis=0)
    return wcat.reshape(BLK, KP, cdim)


def _tc1_body(nf_ref, nx_ref, ny_ref, nz_ref, qx_ref, qy_ref, qz_ref,
              kp_ref, mask_ref, ws_ref, wra_ref, x_ref, y1_ref):
    infl = _influence(nx_ref[...], ny_ref[...], nz_ref[...],
                      qx_ref[...], qy_ref[...], qz_ref[...], kp_ref[...])
    w = _weighted(infl, nf_ref[...], mask_ref[...], CIN)
    acc = jnp.zeros((BLK, C1), dtype=jnp.float32)
    for p in range(K):
        acc = acc + jnp.dot(w[:, p, :], ws_ref[p],
                            preferred_element_type=jnp.float32)
    x = _leaky(acc)
    x_ref[...] = x
    y1_ref[...] = _leaky(jnp.dot(x, wra_ref[...],
                                 preferred_element_type=jnp.float32))


def _tc2_body(nyr_ref, nx_ref, ny_ref, nz_ref, qx_ref, qy_ref, qz_ref,
              kp_ref, mask_ref, x_ref, wrk_ref, wrb_ref,
              wsc_ref, ba_ref, wh1_ref, bh1_ref, wh2_ref, bh2_ref,
              wq_ref, bq_ref, qout_ref, acc_ref):
    i = pl.program_id(0)
    infl = _influence(nx_ref[...], ny_ref[...], nz_ref[...],
                      qx_ref[...], qy_ref[...], qz_ref[...], kp_ref[...])
    w = _weighted(infl, nyr_ref[...], mask_ref[...], CB)
    yacc = jnp.zeros((BLK, CB), dtype=jnp.float32)
    for p in range(K):
        yacc = yacc + jnp.dot(w[:, p, :], wrk_ref[p],
                              preferred_element_type=jnp.float32)
    y = _leaky(yacc)
    y = jnp.dot(y, wrb_ref[...], preferred_element_type=jnp.float32)
    x = x_ref[...]
    x2 = _leaky(y + jnp.dot(x, wsc_ref[...],
                            preferred_element_type=jnp.float32))
    partial = jnp.sum(x2, axis=0, keepdims=True)

    @pl.when(i == 0)
    def _():
        acc_ref[...] = partial

    @pl.when(i > 0)
    def _():
        acc_ref[...] = acc_ref[...] + partial

    @pl.when(i == GRID - 1)
    def _():
        g = acc_ref[...] * (1.0 / N)
        h = jnp.concatenate([g, ba_ref[...]], axis=1)
        h = jnp.maximum(0.0, jnp.dot(h, wh1_ref[...],
                                     preferred_element_type=jnp.float32)
                        + bh1_ref[...])
        h = jnp.maximum(0.0, jnp.dot(h, wh2_ref[...],
                                     preferred_element_type=jnp.float32)
                        + bh2_ref[...])
        qout_ref[...] = jnp.dot(h, wq_ref[...],
                                preferred_element_type=jnp.float32) \
            + bq_ref[...]


def _make_sc_gather_feat_pts():
    mesh = plsc.VectorSubcoreMesh(core_axis_name="c", subcore_axis_name="s",
                                  num_cores=NC, num_subcores=NS)
    return pl.kernel(
        _sc_gather_feat_pts,
        out_type=(
            jax.ShapeDtypeStruct((N * KN, CIN), jnp.float32),
            jax.ShapeDtypeStruct((N * KN,), jnp.float32),
            jax.ShapeDtypeStruct((N * KN,), jnp.float32),
            jax.ShapeDtypeStruct((N * KN,), jnp.float32),
        ),
        mesh=mesh,
        compiler_params=pltpu.CompilerParams(use_tc_tiling_on_sc=False),
        scratch_types=[
            pltpu.VMEM((CHUNK,), jnp.int32),
            pltpu.VMEM((CHUNK,), jnp.int32),
            pltpu.VMEM((CHUNK, CIN), jnp.float32),
            pltpu.VMEM((CHUNK, CIN), jnp.float32),
            pltpu.VMEM((CHUNK,), jnp.float32),
            pltpu.VMEM((CHUNK,), jnp.float32),
            pltpu.VMEM((CHUNK,), jnp.float32),
            pltpu.VMEM((CHUNK,), jnp.float32),
            pltpu.VMEM((CHUNK,), jnp.float32),
            pltpu.VMEM((CHUNK,), jnp.float32),
            pltpu.SemaphoreType.DMA,
            pltpu.SemaphoreType.DMA,
            pltpu.SemaphoreType.DMA,
            pltpu.SemaphoreType.DMA,
        ],
    )


def _make_sc_gather_y():
    mesh = plsc.VectorSubcoreMesh(core_axis_name="c", subcore_axis_name="s",
                                  num_cores=NC, num_subcores=NS)
    return pl.kernel(
        _sc_gather_y,
        out_type=jax.ShapeDtypeStruct((N * KN, CB), jnp.float32),
        mesh=mesh,
        compiler_params=pltpu.CompilerParams(use_tc_tiling_on_sc=False),
        scratch_types=[
            pltpu.VMEM((CHUNK,), jnp.int32),
            pltpu.VMEM((CHUNK,), jnp.int32),
            pltpu.VMEM((CHUNK, CB), jnp.float32),
            pltpu.VMEM((CHUNK, CB), jnp.float32),
            pltpu.SemaphoreType.DMA,
            pltpu.SemaphoreType.DMA,
            pltpu.SemaphoreType.DMA,
            pltpu.SemaphoreType.DMA,
        ],
    )


def _plane_specs():
    return [
        pl.BlockSpec((BLK, KN), lambda i: (i, 0)),
        pl.BlockSpec((BLK, KN), lambda i: (i, 0)),
        pl.BlockSpec((BLK, KN), lambda i: (i, 0)),
        pl.BlockSpec((BLK, 1), lambda i: (i, 0)),
        pl.BlockSpec((BLK, 1), lambda i: (i, 0)),
        pl.BlockSpec((BLK, 1), lambda i: (i, 0)),
        pl.BlockSpec((3, KP, 1), lambda i: (0, 0, 0)),
        pl.BlockSpec((SROW, SCOL), lambda i: (0, 0)),
    ]


def _make_tc1():
    return pl.pallas_call(
        _tc1_body,
        grid=(GRID,),
        in_specs=[pl.BlockSpec((BLK * KN, CIN), lambda i: (i, 0))]
        + _plane_specs()
        + [
            pl.BlockSpec((K, CIN, C1), lambda i: (0, 0, 0)),
            pl.BlockSpec((C1, CB), lambda i: (0, 0)),
        ],
        out_specs=[
            pl.BlockSpec((BLK, C1), lambda i: (i, 0)),
            pl.BlockSpec((BLK, CB), lambda i: (i, 0)),
        ],
        out_shape=[
            jax.ShapeDtypeStruct((N, C1), jnp.float32),
            jax.ShapeDtypeStruct((N, CB), jnp.float32),
        ],
    )


def _make_tc2():
    return pl.pallas_call(
        _tc2_body,
        grid=(GRID,),
        in_specs=[pl.BlockSpec((BLK * KN, CB), lambda i: (i, 0))]
        + _plane_specs()
        + [
            pl.BlockSpec((BLK, C1), lambda i: (i, 0)),
            pl.BlockSpec((K, CB, CB), lambda i: (0, 0, 0)),
            pl.BlockSpec((CB, COUT), lambda i: (0, 0)),
            pl.BlockSpec((C1, COUT), lambda i: (0, 0)),
            pl.BlockSpec((1, A), lambda i: (0, 0)),
            pl.BlockSpec((COUT + A, H), lambda i: (0, 0)),
            pl.BlockSpec((1, H), lambda i: (0, 0)),
            pl.BlockSpec((H, H), lambda i: (0, 0)),
            pl.BlockSpec((1, H), lambda i: (0, 0)),
            pl.BlockSpec((H, 1), lambda i: (0, 0)),
            pl.BlockSpec((1, 1), lambda i: (0, 0)),
        ],
        out_specs=pl.BlockSpec((1, 1), lambda i: (0, 0)),
        out_shape=jax.ShapeDtypeStruct((1, 1), jnp.float32),
        scratch_shapes=[pltpu.VMEM((1, COUT), jnp.float32)],
    )


def _block_diag_mask():
    r = jnp.arange(SROW)[:, None] // KP
    c = jnp.arange(SCOL)[None, :] // KN
    return (r == c).astype(jnp.float32)


def kernel(features, points, neighbors, batch_action, kernel_points,
           W_simple, W_ra, W_rk, W_rb, W_sc, Wh1, bh1, Wh2, bh2, Wq, bq):
    nbr = neighbors.reshape(-1).astype(jnp.int32)
    pts = points.astype(jnp.float32)
    px, py, pz = pts[:, 0], pts[:, 1], pts[:, 2]
    kp = jnp.pad(kernel_points.astype(jnp.float32).T, ((0, 0), (0, 1)),
                 constant_values=1e3).reshape(3, KP, 1)
    mask = _block_diag_mask()

    nf_flat, nx_f, ny_f, nz_f = _make_sc_gather_feat_pts()(
        nbr, features, px, py, pz)
    nx = nx_f.reshape(N, KN)
    nyc = ny_f.reshape(N, KN)
    nz = nz_f.reshape(N, KN)
    qx = px.reshape(N, 1)
    qy = py.reshape(N, 1)
    qz = pz.reshape(N, 1)

    x, y1 = _make_tc1()(nf_flat, nx, nyc, nz, qx, qy, qz, kp, mask,
                        W_simple, W_ra)

    nyr_flat = _make_sc_gather_y()(nbr, y1)

    q = _make_tc2()(nyr_flat, nx, nyc, nz, qx, qy, qz, kp, mask, x,
                    W_rk, W_rb, W_sc, batch_action, Wh1, bh1.reshape(1, H),
                    Wh2, bh2.reshape(1, H), Wq, bq.reshape(1, 1))
    return q


# speedup vs baseline: 1.5335x; 1.0536x over previous
import functools

import jax
import jax.numpy as jnp
from jax import lax
from jax.experimental import pallas as pl
from jax.experimental.pallas import tpu as pltpu
from jax.experimental.pallas import tpu_sc as plsc

N = 10000
NH = N // 2
KN = 32
K = 15
KP = 16
CIN = 128
C1 = 64
CB = 32
COUT = 128
A = 16
H = 256

NC = 2
NS = 16
NW = NC * NS
PER_W = (NH * KN) // NW
CHUNK = 200
N_CHUNKS = PER_W // CHUNK

BLK = 200
GRID = NH // BLK
SB = 8
NSB = BLK // SB
SROW = SB * KP
SCOL = SB * KN


def _sc_gather_feat_pts(nbr_hbm, feat_hbm, px_hbm, py_hbm, pz_hbm,
                        nf_out, nx_out, ny_out, nz_out,
                        idx_a, idx_b, rows_a, rows_b, cx_a, cx_b,
                        cy_a, cy_b, cz_a, cz_b, gs_a, gs_b, ws_a, ws_b):
    wid = lax.axis_index("s") * NC + lax.axis_index("c")
    base = wid * PER_W

    def fire(off, idx_v, rows_v, cx_v, cy_v, cz_v, gs):
        pltpu.sync_copy(nbr_hbm.at[pl.ds(off, CHUNK)], idx_v)
        return (pltpu.async_copy(feat_hbm.at[idx_v], rows_v, gs),
                pltpu.async_copy(px_hbm.at[idx_v], cx_v, gs),
                pltpu.async_copy(py_hbm.at[idx_v], cy_v, gs),
                pltpu.async_copy(pz_hbm.at[idx_v], cz_v, gs))

    def writeback(off, rows_v, cx_v, cy_v, cz_v, ws):
        return (pltpu.async_copy(rows_v, nf_out.at[pl.ds(off, CHUNK)], ws),
                pltpu.async_copy(cx_v, nx_out.at[pl.ds(off, CHUNK)], ws),
                pltpu.async_copy(cy_v, ny_out.at[pl.ds(off, CHUNK)], ws),
                pltpu.async_copy(cz_v, nz_out.at[pl.ds(off, CHUNK)], ws))

    def pair(off0, off1):
        cps_a = fire(off0, idx_a, rows_a, cx_a, cy_a, cz_a, gs_a)
        cps_b = fire(off1, idx_b, rows_b, cx_b, cy_b, cz_b, gs_b)
        for c in cps_a:
            c.wait()
        wb_a = writeback(off0, rows_a, cx_a, cy_a, cz_a, ws_a)
        for c in cps_b:
            c.wait()
        wb_b = writeback(off1, rows_b, cx_b, cy_b, cz_b, ws_b)
        for c in wb_a:
            c.wait()
        for c in wb_b:
            c.wait()

    def body(j, carry):
        off0 = base + (2 * j) * CHUNK
        pair(off0, off0 + CHUNK)
        return carry

    lax.fori_loop(0, N_CHUNKS // 2, body, 0)
    if N_CHUNKS % 2:
        off = base + (N_CHUNKS - 1) * CHUNK
        cps = fire(off, idx_a, rows_a, cx_a, cy_a, cz_a, gs_a)
        for c in cps:
            c.wait()
        for c in writeback(off, rows_a, cx_a, cy_a, cz_a, ws_a):
            c.wait()


def _sc_gather_y(nbr_hbm, y_hbm, ny_out, idx_a, idx_b, rows_a, rows_b,
                 gs_a, gs_b, ws_a, ws_b):
    wid = lax.axis_index("s") * NC + lax.axis_index("c")
    base = wid * PER_W

    def pair(off0, off1):
        pltpu.sync_copy(nbr_hbm.at[pl.ds(off0, CHUNK)], idx_a)
        cp_a = pltpu.async_copy(y_hbm.at[idx_a], rows_a, gs_a)
        pltpu.sync_copy(nbr_hbm.at[pl.ds(off1, CHUNK)], idx_b)
        cp_b = pltpu.async_copy(y_hbm.at[idx_b], rows_b, gs_b)
        cp_a.wait()
        wb_a = pltpu.async_copy(rows_a, ny_out.at[pl.ds(off0, CHUNK)], ws_a)
        cp_b.wait()
        wb_b = pltpu.async_copy(rows_b, ny_out.at[pl.ds(off1, CHUNK)], ws_b)
        wb_a.wait()
        wb_b.wait()

    def body(j, carry):
        off0 = base + (2 * j) * CHUNK
        pair(off0, off0 + CHUNK)
        return carry

    lax.fori_loop(0, N_CHUNKS // 2, body, 0)
    if N_CHUNKS % 2:
        off = base + (N_CHUNKS - 1) * CHUNK
        pltpu.sync_copy(nbr_hbm.at[pl.ds(off, CHUNK)], idx_a)
        pltpu.async_copy(y_hbm.at[idx_a], rows_a, gs_a).wait()
        pltpu.async_copy(rows_a, ny_out.at[pl.ds(off, CHUNK)], ws_a).wait()


def _leaky(v):
    return jnp.where(v >= 0, v, 0.1 * v)


def _influence(nx, ny, nz, qx, qy, qz, kp):
    dx = (nx - qx)[:, None, :] - kp[0][None, :, :]
    dy = (ny - qy)[:, None, :] - kp[1][None, :, :]
    dz = (nz - qz)[:, None, :] - kp[2][None, :, :]
    d2 = dx * dx + dy * dy + dz * dz
    dist = jnp.sqrt(d2 + 1e-12)
    return jnp.maximum(0.0, 1.0 - dist)


def _weighted(infl, rows, mask, cdim):
    tiles = infl.reshape(NSB, SROW, KN)
    tiles = jnp.concatenate([tiles] * SB, axis=-1)
    s_all = tiles * mask[None]
    rows_sb = rows.reshape(NSB, SCOL, cdim)
    outs = [jnp.dot(s_all[j], rows_sb[j], preferred_element_type=jnp.float32)
            for j in range(NSB)]
    wcat = jnp.concatenate(outs, axis=0)
    return wcat.reshape(BLK, KP, cdim)


def _tc1_body(nf_ref, nx_ref, ny_ref, nz_ref, qx_ref, qy_ref, qz_ref,
              kp_ref, mask_ref, ws_ref, wra_ref, x_ref, y1_ref):
    infl = _influence(nx_ref[...], ny_ref[...], nz_ref[...],
                      qx_ref[...], qy_ref[...], qz_ref[...], kp_ref[...])
    w = _weighted(infl, nf_ref[...], mask_ref[...], CIN)
    acc = jnp.zeros((BLK, C1), dtype=jnp.float32)
    for p in range(K):
        acc = acc + jnp.dot(w[:, p, :], ws_ref[p],
                            preferred_element_type=jnp.float32)
    x = _leaky(acc)
    x_ref[...] = x
    y1_ref[...] = _leaky(jnp.dot(x, wra_ref[...],
                                 preferred_element_type=jnp.float32))


def _tc2_body(is_last, nyr_ref, nx_ref, ny_ref, nz_ref, qx_ref, qy_ref,
              qz_ref, kp_ref, mask_ref, x_ref, wrk_ref, wrb_ref,
              wsc_ref, gprev_ref, ba_ref, wh1_ref, bh1_ref, wh2_ref,
              bh2_ref, wq_ref, bq_ref, out_ref, acc_ref):
    i = pl.program_id(0)
    infl = _influence(nx_ref[...], ny_ref[...], nz_ref[...],
                      qx_ref[...], qy_ref[...], qz_ref[...], kp_ref[...])
    w = _weighted(infl, nyr_ref[...], mask_ref[...], CB)
    yacc = jnp.zeros((BLK, CB), dtype=jnp.float32)
    for p in range(K):
        yacc = yacc + jnp.dot(w[:, p, :], wrk_ref[p],
                              preferred_element_type=jnp.float32)
    y = _leaky(yacc)
    y = jnp.dot(y, wrb_ref[...], preferred_element_type=jnp.float32)
    x = x_ref[...]
    x2 = _leaky(y + jnp.dot(x, wsc_ref[...],
                            preferred_element_type=jnp.float32))
    partial = jnp.sum(x2, axis=0, keepdims=True)

    @pl.when(i == 0)
    def _():
        acc_ref[...] = gprev_ref[...] + partial

    @pl.when(i > 0)
    def _():
        acc_ref[...] = acc_ref[...] + partial

    if not is_last:
        @pl.when(i == GRID - 1)
        def _():
            out_ref[...] = acc_ref[...]
    else:
        @pl.when(i == GRID - 1)
        def _():
            g = acc_ref[...] * (1.0 / N)
            h = jnp.concatenate([g, ba_ref[...]], axis=1)
            h = jnp.maximum(0.0, jnp.dot(h, wh1_ref[...],
                                         preferred_element_type=jnp.float32)
                            + bh1_ref[...])
            h = jnp.maximum(0.0, jnp.dot(h, wh2_ref[...],
                                         preferred_element_type=jnp.float32)
                            + bh2_ref[...])
            out_ref[...] = jnp.dot(h, wq_ref[...],
                                   preferred_element_type=jnp.float32) \
                + bq_ref[...]


def _make_sc_gather_feat_pts():
    mesh = plsc.VectorSubcoreMesh(core_axis_name="c", subcore_axis_name="s",
                                  num_cores=NC, num_subcores=NS)
    return pl.kernel(
        _sc_gather_feat_pts,
        out_type=(
            jax.ShapeDtypeStruct((NH * KN, CIN), jnp.float32),
            jax.ShapeDtypeStruct((NH * KN,), jnp.float32),
            jax.ShapeDtypeStruct((NH * KN,), jnp.float32),
            jax.ShapeDtypeStruct((NH * KN,), jnp.float32),
        ),
        mesh=mesh,
        compiler_params=pltpu.CompilerParams(use_tc_tiling_on_sc=False),
        scratch_types=[
            pltpu.VMEM((CHUNK,), jnp.int32),
            pltpu.VMEM((CHUNK,), jnp.int32),
            pltpu.VMEM((CHUNK, CIN), jnp.float32),
            pltpu.VMEM((CHUNK, CIN), jnp.float32),
            pltpu.VMEM((CHUNK,), jnp.float32),
            pltpu.VMEM((CHUNK,), jnp.float32),
            pltpu.VMEM((CHUNK,), jnp.float32),
            pltpu.VMEM((CHUNK,), jnp.float32),
            pltpu.VMEM((CHUNK,), jnp.float32),
            pltpu.VMEM((CHUNK,), jnp.float32),
            pltpu.SemaphoreType.DMA,
            pltpu.SemaphoreType.DMA,
            pltpu.SemaphoreType.DMA,
            pltpu.SemaphoreType.DMA,
        ],
    )


def _make_sc_gather_y():
    mesh = plsc.VectorSubcoreMesh(core_axis_name="c", subcore_axis_name="s",
                                  num_cores=NC, num_subcores=NS)
    return pl.kernel(
        _sc_gather_y,
        out_type=jax.ShapeDtypeStruct((NH * KN, CB), jnp.float32),
        mesh=mesh,
        compiler_params=pltpu.CompilerParams(use_tc_tiling_on_sc=False),
        scratch_types=[
            pltpu.VMEM((CHUNK,), jnp.int32),
            pltpu.VMEM((CHUNK,), jnp.int32),
            pltpu.VMEM((CHUNK, CB), jnp.float32),
            pltpu.VMEM((CHUNK, CB), jnp.float32),
            pltpu.SemaphoreType.DMA,
            pltpu.SemaphoreType.DMA,
            pltpu.SemaphoreType.DMA,
            pltpu.SemaphoreType.DMA,
        ],
    )


def _plane_specs():
    return [
        pl.BlockSpec((BLK, KN), lambda i: (i, 0)),
        pl.BlockSpec((BLK, KN), lambda i: (i, 0)),
        pl.BlockSpec((BLK, KN), lambda i: (i, 0)),
        pl.BlockSpec((BLK, 1), lambda i: (i, 0)),
        pl.BlockSpec((BLK, 1), lambda i: (i, 0)),
        pl.BlockSpec((BLK, 1), lambda i: (i, 0)),
        pl.BlockSpec((3, KP, 1), lambda i: (0, 0, 0)),
        pl.BlockSpec((SROW, SCOL), lambda i: (0, 0)),
    ]


def _make_tc1():
    return pl.pallas_call(
        _tc1_body,
        grid=(GRID,),
        in_specs=[pl.BlockSpec((BLK * KN, CIN), lambda i: (i, 0))]
        + _plane_specs()
        + [
            pl.BlockSpec((K, CIN, C1), lambda i: (0, 0, 0)),
            pl.BlockSpec((C1, CB), lambda i: (0, 0)),
        ],
        out_specs=[
            pl.BlockSpec((BLK, C1), lambda i: (i, 0)),
            pl.BlockSpec((BLK, CB), lambda i: (i, 0)),
        ],
        out_shape=[
            jax.ShapeDtypeStruct((NH, C1), jnp.float32),
            jax.ShapeDtypeStruct((NH, CB), jnp.float32),
        ],
    )


def _make_tc2(is_last):
    return pl.pallas_call(
        functools.partial(_tc2_body, is_last),
        grid=(GRID,),
        in_specs=[pl.BlockSpec((BLK * KN, CB), lambda i: (i, 0))]
        + _plane_specs()
        + [
            pl.BlockSpec((BLK, C1), lambda i: (i, 0)),
            pl.BlockSpec((K, CB, CB), lambda i: (0, 0, 0)),
            pl.BlockSpec((CB, COUT), lambda i: (0, 0)),
            pl.BlockSpec((C1, COUT), lambda i: (0, 0)),
            pl.BlockSpec((1, COUT), lambda i: (0, 0)),
            pl.BlockSpec((1, A), lambda i: (0, 0)),
            pl.BlockSpec((COUT + A, H), lambda i: (0, 0)),
            pl.BlockSpec((1, H), lambda i: (0, 0)),
            pl.BlockSpec((H, H), lambda i: (0, 0)),
            pl.BlockSpec((1, H), lambda i: (0, 0)),
            pl.BlockSpec((H, 1), lambda i: (0, 0)),
            pl.BlockSpec((1, 1), lambda i: (0, 0)),
        ],
        out_specs=pl.BlockSpec(
            (1, 1) if is_last else (1, COUT), lambda i: (0, 0)),
        out_shape=jax.ShapeDtypeStruct(
            (1, 1) if is_last else (1, COUT), jnp.float32),
        scratch_shapes=[pltpu.VMEM((1, COUT), jnp.float32)],
    )


def _block_diag_mask():
    r = jnp.arange(SROW)[:, None] // KP
    c = jnp.arange(SCOL)[None, :] // KN
    return (r == c).astype(jnp.float32)


def kernel(features, points, neighbors, batch_action, kernel_points,
           W_simple, W_ra, W_rk, W_rb, W_sc, Wh1, bh1, Wh2, bh2, Wq, bq):
    nbr = neighbors.reshape(-1).astype(jnp.int32)
    pts = points.astype(jnp.float32)
    px, py, pz = pts[:, 0], pts[:, 1], pts[:, 2]
    kp = jnp.pad(kernel_points.astype(jnp.float32).T, ((0, 0), (0, 1)),
                 constant_values=1e3).reshape(3, KP, 1)
    mask = _block_diag_mask()

    sc1 = _make_sc_gather_feat_pts()
    tc1 = _make_tc1()
    scy = _make_sc_gather_y()

    halves = []
    for h_i in range(2):
        nbr_h = lax.slice_in_dim(nbr, h_i * NH * KN, (h_i + 1) * NH * KN)
        nf_flat, nx_f, ny_f, nz_f = sc1(nbr_h, features, px, py, pz)
        halves.append((
            nbr_h, nf_flat,
            nx_f.reshape(NH, KN), ny_f.reshape(NH, KN), nz_f.reshape(NH, KN),
            lax.slice_in_dim(px, h_i * NH, (h_i + 1) * NH).reshape(NH, 1),
            lax.slice_in_dim(py, h_i * NH, (h_i + 1) * NH).reshape(NH, 1),
            lax.slice_in_dim(pz, h_i * NH, (h_i + 1) * NH).reshape(NH, 1),
        ))

    xs, y1s = [], []
    for (nbr_h, nf_flat, nx, nyc, nz, qx, qy, qz) in halves:
        x_h, y1_h = tc1(nf_flat, nx, nyc, nz, qx, qy, qz, kp, mask,
                        W_simple, W_ra)
        xs.append(x_h)
        y1s.append(y1_h)
    y1 = jnp.concatenate(y1s, axis=0)

    nys = [scy(h[0], y1) for h in halves]

    gprev = jnp.zeros((1, COUT), jnp.float32)
    out = None
    for h_i in range(2):
        (nbr_h, nf_flat, nx, nyc, nz, qx, qy, qz) = halves[h_i]
        out = _make_tc2(h_i == 1)(
            nys[h_i], nx, nyc, nz, qx, qy, qz, kp, mask, xs[h_i],
            W_rk, W_rb, W_sc, gprev, batch_action, Wh1, bh1.reshape(1, H),
            Wh2, bh2.reshape(1, H), Wq, bq.reshape(1, 1))
        gprev = out
    return out


# speedup vs baseline: 1.5344x; 1.0006x over previous
import functools

import jax
import jax.numpy as jnp
from jax import lax
from jax.experimental import pallas as pl
from jax.experimental.pallas import tpu as pltpu
from jax.experimental.pallas import tpu_sc as plsc

N = 10000
N_SPLIT = 5
NH = N // N_SPLIT
KN = 32
K = 15
KP = 16
CIN = 128
C1 = 64
CB = 32
COUT = 128
A = 16
H = 256

NC = 2
NS = 16
NW = NC * NS
PER_W = (NH * KN) // NW
CHUNK = 200
N_CHUNKS = PER_W // CHUNK

BLK = 200
GRID = NH // BLK
SB = 8
NSB = BLK // SB
SROW = SB * KP
SCOL = SB * KN


def _sc_gather_feat_pts(nbr_hbm, feat_hbm, px_hbm, py_hbm, pz_hbm,
                        nf_out, nx_out, ny_out, nz_out,
                        idx_a, idx_b, rows_a, rows_b, cx_a, cx_b,
                        cy_a, cy_b, cz_a, cz_b, gs_a, gs_b, ws_a, ws_b):
    wid = lax.axis_index("s") * NC + lax.axis_index("c")
    base = wid * PER_W

    def fire(off, idx_v, rows_v, cx_v, cy_v, cz_v, gs):
        pltpu.sync_copy(nbr_hbm.at[pl.ds(off, CHUNK)], idx_v)
        return (pltpu.async_copy(feat_hbm.at[idx_v], rows_v, gs),
                pltpu.async_copy(px_hbm.at[idx_v], cx_v, gs),
                pltpu.async_copy(py_hbm.at[idx_v], cy_v, gs),
                pltpu.async_copy(pz_hbm.at[idx_v], cz_v, gs))

    def writeback(off, rows_v, cx_v, cy_v, cz_v, ws):
        return (pltpu.async_copy(rows_v, nf_out.at[pl.ds(off, CHUNK)], ws),
                pltpu.async_copy(cx_v, nx_out.at[pl.ds(off, CHUNK)], ws),
                pltpu.async_copy(cy_v, ny_out.at[pl.ds(off, CHUNK)], ws),
                pltpu.async_copy(cz_v, nz_out.at[pl.ds(off, CHUNK)], ws))

    def pair(off0, off1):
        cps_a = fire(off0, idx_a, rows_a, cx_a, cy_a, cz_a, gs_a)
        cps_b = fire(off1, idx_b, rows_b, cx_b, cy_b, cz_b, gs_b)
        for c in cps_a:
            c.wait()
        wb_a = writeback(off0, rows_a, cx_a, cy_a, cz_a, ws_a)
        for c in cps_b:
            c.wait()
        wb_b = writeback(off1, rows_b, cx_b, cy_b, cz_b, ws_b)
        for c in wb_a:
            c.wait()
        for c in wb_b:
            c.wait()

    def body(j, carry):
        off0 = base + (2 * j) * CHUNK
        pair(off0, off0 + CHUNK)
        return carry

    lax.fori_loop(0, N_CHUNKS // 2, body, 0)
    if N_CHUNKS % 2:
        off = base + (N_CHUNKS - 1) * CHUNK
        cps = fire(off, idx_a, rows_a, cx_a, cy_a, cz_a, gs_a)
        for c in cps:
            c.wait()
        for c in writeback(off, rows_a, cx_a, cy_a, cz_a, ws_a):
            c.wait()


def _sc_gather_y(nbr_hbm, y_hbm, ny_out, idx_a, idx_b, rows_a, rows_b,
                 gs_a, gs_b, ws_a, ws_b):
    wid = lax.axis_index("s") * NC + lax.axis_index("c")
    base = wid * PER_W

    def pair(off0, off1):
        pltpu.sync_copy(nbr_hbm.at[pl.ds(off0, CHUNK)], idx_a)
        cp_a = pltpu.async_copy(y_hbm.at[idx_a], rows_a, gs_a)
        pltpu.sync_copy(nbr_hbm.at[pl.ds(off1, CHUNK)], idx_b)
        cp_b = pltpu.async_copy(y_hbm.at[idx_b], rows_b, gs_b)
        cp_a.wait()
        wb_a = pltpu.async_copy(rows_a, ny_out.at[pl.ds(off0, CHUNK)], ws_a)
        cp_b.wait()
        wb_b = pltpu.async_copy(rows_b, ny_out.at[pl.ds(off1, CHUNK)], ws_b)
        wb_a.wait()
        wb_b.wait()

    def body(j, carry):
        off0 = base + (2 * j) * CHUNK
        pair(off0, off0 + CHUNK)
        return carry

    lax.fori_loop(0, N_CHUNKS // 2, body, 0)
    if N_CHUNKS % 2:
        off = base + (N_CHUNKS - 1) * CHUNK
        pltpu.sync_copy(nbr_hbm.at[pl.ds(off, CHUNK)], idx_a)
        pltpu.async_copy(y_hbm.at[idx_a], rows_a, gs_a).wait()
        pltpu.async_copy(rows_a, ny_out.at[pl.ds(off, CHUNK)], ws_a).wait()


def _leaky(v):
    return jnp.where(v >= 0, v, 0.1 * v)


def _influence(nx, ny, nz, qx, qy, qz, kp):
    dx = (nx - qx)[:, None, :] - kp[0][None, :, :]
    dy = (ny - qy)[:, None, :] - kp[1][None, :, :]
    dz = (nz - qz)[:, None, :] - kp[2][None, :, :]
    d2 = dx * dx + dy * dy + dz * dz
    dist = jnp.sqrt(d2 + 1e-12)
    return jnp.maximum(0.0, 1.0 - dist)


def _weighted(infl, rows, mask, cdim):
    tiles = infl.reshape(NSB, SROW, KN)
    tiles = jnp.concatenate([tiles] * SB, axis=-1)
    s_all = tiles * mask[None]
    rows_sb = rows.reshape(NSB, SCOL, cdim)
    outs = [jnp.dot(s_all[j], rows_sb[j], preferred_element_type=jnp.float32)
            for j in range(NSB)]
    wcat = jnp.concatenate(outs, axis=0)
    return wcat.reshape(BLK, KP, cdim)


def _tc1_body(nf_ref, nx_ref, ny_ref, nz_ref, qx_ref, qy_ref, qz_ref,
              kp_ref, mask_ref, ws_ref, wra_ref, x_ref, y1_ref):
    infl = _influence(nx_ref[...], ny_ref[...], nz_ref[...],
                      qx_ref[...], qy_ref[...], qz_ref[...], kp_ref[...])
    w = _weighted(infl, nf_ref[...], mask_ref[...], CIN)
    acc = jnp.zeros((BLK, C1), dtype=jnp.float32)
    for p in range(K):
        acc = acc + jnp.dot(w[:, p, :], ws_ref[p],
                            preferred_element_type=jnp.float32)
    x = _leaky(acc)
    x_ref[...] = x
    y1_ref[...] = _leaky(jnp.dot(x, wra_ref[...],
                                 preferred_element_type=jnp.float32))


def _tc2_body(is_last, nyr_ref, nx_ref, ny_ref, nz_ref, qx_ref, qy_ref,
              qz_ref, kp_ref, mask_ref, x_ref, wrk_ref, wrb_ref,
              wsc_ref, gprev_ref, ba_ref, wh1_ref, bh1_ref, wh2_ref,
              bh2_ref, wq_ref, bq_ref, out_ref, acc_ref):
    i = pl.program_id(0)
    infl = _influence(nx_ref[...], ny_ref[...], nz_ref[...],
                      qx_ref[...], qy_ref[...], qz_ref[...], kp_ref[...])
    w = _weighted(infl, nyr_ref[...], mask_ref[...], CB)
    yacc = jnp.zeros((BLK, CB), dtype=jnp.float32)
    for p in range(K):
        yacc = yacc + jnp.dot(w[:, p, :], wrk_ref[p],
                              preferred_element_type=jnp.float32)
    y = _leaky(yacc)
    y = jnp.dot(y, wrb_ref[...], preferred_element_type=jnp.float32)
    x = x_ref[...]
    x2 = _leaky(y + jnp.dot(x, wsc_ref[...],
                            preferred_element_type=jnp.float32))
    partial = jnp.sum(x2, axis=0, keepdims=True)

    @pl.when(i == 0)
    def _():
        acc_ref[...] = gprev_ref[...] + partial

    @pl.when(i > 0)
    def _():
        acc_ref[...] = acc_ref[...] + partial

    if not is_last:
        @pl.when(i == GRID - 1)
        def _():
            out_ref[...] = acc_ref[...]
    else:
        @pl.when(i == GRID - 1)
        def _():
            g = acc_ref[...] * (1.0 / N)
            h = jnp.concatenate([g, ba_ref[...]], axis=1)
            h = jnp.maximum(0.0, jnp.dot(h, wh1_ref[...],
                                         preferred_element_type=jnp.float32)
                            + bh1_ref[...])
            h = jnp.maximum(0.0, jnp.dot(h, wh2_ref[...],
                                         preferred_element_type=jnp.float32)
                            + bh2_ref[...])
            out_ref[...] = jnp.dot(h, wq_ref[...],
                                   preferred_element_type=jnp.float32) \
                + bq_ref[...]


def _make_sc_gather_feat_pts():
    mesh = plsc.VectorSubcoreMesh(core_axis_name="c", subcore_axis_name="s",
                                  num_cores=NC, num_subcores=NS)
    return pl.kernel(
        _sc_gather_feat_pts,
        out_type=(
            jax.ShapeDtypeStruct((NH * KN, CIN), jnp.float32),
            jax.ShapeDtypeStruct((NH * KN,), jnp.float32),
            jax.ShapeDtypeStruct((NH * KN,), jnp.float32),
            jax.ShapeDtypeStruct((NH * KN,), jnp.float32),
        ),
        mesh=mesh,
        compiler_params=pltpu.CompilerParams(use_tc_tiling_on_sc=False),
        scratch_types=[
            pltpu.VMEM((CHUNK,), jnp.int32),
            pltpu.VMEM((CHUNK,), jnp.int32),
            pltpu.VMEM((CHUNK, CIN), jnp.float32),
            pltpu.VMEM((CHUNK, CIN), jnp.float32),
            pltpu.VMEM((CHUNK,), jnp.float32),
            pltpu.VMEM((CHUNK,), jnp.float32),
            pltpu.VMEM((CHUNK,), jnp.float32),
            pltpu.VMEM((CHUNK,), jnp.float32),
            pltpu.VMEM((CHUNK,), jnp.float32),
            pltpu.VMEM((CHUNK,), jnp.float32),
            pltpu.SemaphoreType.DMA,
            pltpu.SemaphoreType.DMA,
            pltpu.SemaphoreType.DMA,
            pltpu.SemaphoreType.DMA,
        ],
    )


def _make_sc_gather_y():
    mesh = plsc.VectorSubcoreMesh(core_axis_name="c", subcore_axis_name="s",
                                  num_cores=NC, num_subcores=NS)
    return pl.kernel(
        _sc_gather_y,
        out_type=jax.ShapeDtypeStruct((NH * KN, CB), jnp.float32),
        mesh=mesh,
        compiler_params=pltpu.CompilerParams(use_tc_tiling_on_sc=False),
        scratch_types=[
            pltpu.VMEM((CHUNK,), jnp.int32),
            pltpu.VMEM((CHUNK,), jnp.int32),
            pltpu.VMEM((CHUNK, CB), jnp.float32),
            pltpu.VMEM((CHUNK, CB), jnp.float32),
            pltpu.SemaphoreType.DMA,
            pltpu.SemaphoreType.DMA,
            pltpu.SemaphoreType.DMA,
            pltpu.SemaphoreType.DMA,
        ],
    )


def _plane_specs():
    return [
        pl.BlockSpec((BLK, KN), lambda i: (i, 0)),
        pl.BlockSpec((BLK, KN), lambda i: (i, 0)),
        pl.BlockSpec((BLK, KN), lambda i: (i, 0)),
        pl.BlockSpec((BLK, 1), lambda i: (i, 0)),
        pl.BlockSpec((BLK, 1), lambda i: (i, 0)),
        pl.BlockSpec((BLK, 1), lambda i: (i, 0)),
        pl.BlockSpec((3, KP, 1), lambda i: (0, 0, 0)),
        pl.BlockSpec((SROW, SCOL), lambda i: (0, 0)),
    ]


def _make_tc1():
    return pl.pallas_call(
        _tc1_body,
        grid=(GRID,),
        in_specs=[pl.BlockSpec((BLK * KN, CIN), lambda i: (i, 0))]
        + _plane_specs()
        + [
            pl.BlockSpec((K, CIN, C1), lambda i: (0, 0, 0)),
            pl.BlockSpec((C1, CB), lambda i: (0, 0)),
        ],
        out_specs=[
            pl.BlockSpec((BLK, C1), lambda i: (i, 0)),
            pl.BlockSpec((BLK, CB), lambda i: (i, 0)),
        ],
        out_shape=[
            jax.ShapeDtypeStruct((NH, C1), jnp.float32),
            jax.ShapeDtypeStruct((NH, CB), jnp.float32),
        ],
    )


def _make_tc2(is_last):
    return pl.pallas_call(
        functools.partial(_tc2_body, is_last),
        grid=(GRID,),
        in_specs=[pl.BlockSpec((BLK * KN, CB), lambda i: (i, 0))]
        + _plane_specs()
        + [
            pl.BlockSpec((BLK, C1), lambda i: (i, 0)),
            pl.BlockSpec((K, CB, CB), lambda i: (0, 0, 0)),
            pl.BlockSpec((CB, COUT), lambda i: (0, 0)),
            pl.BlockSpec((C1, COUT), lambda i: (0, 0)),
            pl.BlockSpec((1, COUT), lambda i: (0, 0)),
            pl.BlockSpec((1, A), lambda i: (0, 0)),
            pl.BlockSpec((COUT + A, H), lambda i: (0, 0)),
            pl.BlockSpec((1, H), lambda i: (0, 0)),
            pl.BlockSpec((H, H), lambda i: (0, 0)),
            pl.BlockSpec((1, H), lambda i: (0, 0)),
            pl.BlockSpec((H, 1), lambda i: (0, 0)),
            pl.BlockSpec((1, 1), lambda i: (0, 0)),
        ],
        out_specs=pl.BlockSpec(
            (1, 1) if is_last else (1, COUT), lambda i: (0, 0)),
        out_shape=jax.ShapeDtypeStruct(
            (1, 1) if is_last else (1, COUT), jnp.float32),
        scratch_shapes=[pltpu.VMEM((1, COUT), jnp.float32)],
    )


def _block_diag_mask():
    r = jnp.arange(SROW)[:, None] // KP
    c = jnp.arange(SCOL)[None, :] // KN
    return (r == c).astype(jnp.float32)


def kernel(features, points, neighbors, batch_action, kernel_points,
           W_simple, W_ra, W_rk, W_rb, W_sc, Wh1, bh1, Wh2, bh2, Wq, bq):
    nbr = neighbors.reshape(-1).astype(jnp.int32)
    pts = points.astype(jnp.float32)
    px, py, pz = pts[:, 0], pts[:, 1], pts[:, 2]
    kp = jnp.pad(kernel_points.astype(jnp.float32).T, ((0, 0), (0, 1)),
                 constant_values=1e3).reshape(3, KP, 1)
    mask = _block_diag_mask()

    sc1 = _make_sc_gather_feat_pts()
    tc1 = _make_tc1()
    scy = _make_sc_gather_y()

    halves = []
    for h_i in range(N_SPLIT):
        nbr_h = lax.slice_in_dim(nbr, h_i * NH * KN, (h_i + 1) * NH * KN)
        nf_flat, nx_f, ny_f, nz_f = sc1(nbr_h, features, px, py, pz)
        halves.append((
            nbr_h, nf_flat,
            nx_f.reshape(NH, KN), ny_f.reshape(NH, KN), nz_f.reshape(NH, KN),
            lax.slice_in_dim(px, h_i * NH, (h_i + 1) * NH).reshape(NH, 1),
            lax.slice_in_dim(py, h_i * NH, (h_i + 1) * NH).reshape(NH, 1),
            lax.slice_in_dim(pz, h_i * NH, (h_i + 1) * NH).reshape(NH, 1),
        ))

    xs, y1s = [], []
    for (nbr_h, nf_flat, nx, nyc, nz, qx, qy, qz) in halves:
        x_h, y1_h = tc1(nf_flat, nx, nyc, nz, qx, qy, qz, kp, mask,
                        W_simple, W_ra)
        xs.append(x_h)
        y1s.append(y1_h)
    y1 = jnp.concatenate(y1s, axis=0)

    nys = [scy(h[0], y1) for h in halves]

    gprev = jnp.zeros((1, COUT), jnp.float32)
    out = None
    for h_i in range(N_SPLIT):
        (nbr_h, nf_flat, nx, nyc, nz, qx, qy, qz) = halves[h_i]
        out = _make_tc2(h_i == N_SPLIT - 1)(
            nys[h_i], nx, nyc, nz, qx, qy, qz, kp, mask, xs[h_i],
            W_rk, W_rb, W_sc, gprev, batch_action, Wh1, bh1.reshape(1, H),
            Wh2, bh2.reshape(1, H), Wq, bq.reshape(1, 1))
        gprev = out
    return out


# speedup vs baseline: 1.5713x; 1.0240x over previous
import functools

import jax
import jax.numpy as jnp
from jax import lax
from jax.experimental import pallas as pl
from jax.experimental.pallas import tpu as pltpu
from jax.experimental.pallas import tpu_sc as plsc

N = 10000
N_SPLIT = 2
NH = N // N_SPLIT
KN = 32
K = 15
KP = 16
CIN = 128
C1 = 64
CB = 32
COUT = 128
A = 16
H = 256

NC = 2
NS = 16
NW = NC * NS
PER_W = (NH * KN) // NW
CHUNK = 400
N_CHUNKS = PER_W // CHUNK

BLK = 200
GRID = NH // BLK
SB = 8
NSB = BLK // SB
SROW = SB * KP
SCOL = SB * KN


def _sc_gather_feat_pts(nbr_hbm, feat_hbm, px_hbm, py_hbm, pz_hbm,
                        nf_out, nx_out, ny_out, nz_out,
                        idx_a, idx_b, rows_a, rows_b, cx_a, cx_b,
                        cy_a, cy_b, cz_a, cz_b, gs_a, gs_b, ws_a, ws_b):
    wid = lax.axis_index("s") * NC + lax.axis_index("c")
    base = wid * PER_W

    def fire(off, idx_v, rows_v, cx_v, cy_v, cz_v, gs):
        pltpu.sync_copy(nbr_hbm.at[pl.ds(off, CHUNK)], idx_v)
        return (pltpu.async_copy(feat_hbm.at[idx_v], rows_v, gs),
                pltpu.async_copy(px_hbm.at[idx_v], cx_v, gs),
                pltpu.async_copy(py_hbm.at[idx_v], cy_v, gs),
                pltpu.async_copy(pz_hbm.at[idx_v], cz_v, gs))

    def writeback(off, rows_v, cx_v, cy_v, cz_v, ws):
        return (pltpu.async_copy(rows_v, nf_out.at[pl.ds(off, CHUNK)], ws),
                pltpu.async_copy(cx_v, nx_out.at[pl.ds(off, CHUNK)], ws),
                pltpu.async_copy(cy_v, ny_out.at[pl.ds(off, CHUNK)], ws),
                pltpu.async_copy(cz_v, nz_out.at[pl.ds(off, CHUNK)], ws))

    def pair(off0, off1):
        cps_a = fire(off0, idx_a, rows_a, cx_a, cy_a, cz_a, gs_a)
        cps_b = fire(off1, idx_b, rows_b, cx_b, cy_b, cz_b, gs_b)
        for c in cps_a:
            c.wait()
        wb_a = writeback(off0, rows_a, cx_a, cy_a, cz_a, ws_a)
        for c in cps_b:
            c.wait()
        wb_b = writeback(off1, rows_b, cx_b, cy_b, cz_b, ws_b)
        for c in wb_a:
            c.wait()
        for c in wb_b:
            c.wait()

    def body(j, carry):
        off0 = base + (2 * j) * CHUNK
        pair(off0, off0 + CHUNK)
        return carry

    lax.fori_loop(0, N_CHUNKS // 2, body, 0)
    if N_CHUNKS % 2:
        off = base + (N_CHUNKS - 1) * CHUNK
        cps = fire(off, idx_a, rows_a, cx_a, cy_a, cz_a, gs_a)
        for c in cps:
            c.wait()
        for c in writeback(off, rows_a, cx_a, cy_a, cz_a, ws_a):
            c.wait()


def _sc_gather_y(nbr_hbm, y_hbm, ny_out, idx_a, idx_b, rows_a, rows_b,
                 gs_a, gs_b, ws_a, ws_b):
    wid = lax.axis_index("s") * NC + lax.axis_index("c")
    base = wid * PER_W

    def pair(off0, off1):
        pltpu.sync_copy(nbr_hbm.at[pl.ds(off0, CHUNK)], idx_a)
        cp_a = pltpu.async_copy(y_hbm.at[idx_a], rows_a, gs_a)
        pltpu.sync_copy(nbr_hbm.at[pl.ds(off1, CHUNK)], idx_b)
        cp_b = pltpu.async_copy(y_hbm.at[idx_b], rows_b, gs_b)
        cp_a.wait()
        wb_a = pltpu.async_copy(rows_a, ny_out.at[pl.ds(off0, CHUNK)], ws_a)
        cp_b.wait()
        wb_b = pltpu.async_copy(rows_b, ny_out.at[pl.ds(off1, CHUNK)], ws_b)
        wb_a.wait()
        wb_b.wait()

    def body(j, carry):
        off0 = base + (2 * j) * CHUNK
        pair(off0, off0 + CHUNK)
        return carry

    lax.fori_loop(0, N_CHUNKS // 2, body, 0)
    if N_CHUNKS % 2:
        off = base + (N_CHUNKS - 1) * CHUNK
        pltpu.sync_copy(nbr_hbm.at[pl.ds(off, CHUNK)], idx_a)
        pltpu.async_copy(y_hbm.at[idx_a], rows_a, gs_a).wait()
        pltpu.async_copy(rows_a, ny_out.at[pl.ds(off, CHUNK)], ws_a).wait()


def _leaky(v):
    return jnp.where(v >= 0, v, 0.1 * v)


def _influence(nx, ny, nz, qx, qy, qz, kp):
    dx = (nx - qx)[:, None, :] - kp[0][None, :, :]
    dy = (ny - qy)[:, None, :] - kp[1][None, :, :]
    dz = (nz - qz)[:, None, :] - kp[2][None, :, :]
    d2 = dx * dx + dy * dy + dz * dz
    dist = jnp.sqrt(d2 + 1e-12)
    return jnp.maximum(0.0, 1.0 - dist)


def _weighted(infl, rows, mask, cdim):
    tiles = infl.reshape(NSB, SROW, KN)
    tiles = jnp.concatenate([tiles] * SB, axis=-1)
    s_all = tiles * mask[None]
    rows_sb = rows.reshape(NSB, SCOL, cdim)
    outs = [jnp.dot(s_all[j], rows_sb[j], preferred_element_type=jnp.float32)
            for j in range(NSB)]
    wcat = jnp.concatenate(outs, axis=0)
    return wcat.reshape(BLK, KP, cdim)


def _tc1_body(nf_ref, nx_ref, ny_ref, nz_ref, qx_ref, qy_ref, qz_ref,
              kp_ref, mask_ref, ws_ref, wra_ref, x_ref, y1_ref):
    infl = _influence(nx_ref[...], ny_ref[...], nz_ref[...],
                      qx_ref[...], qy_ref[...], qz_ref[...], kp_ref[...])
    w = _weighted(infl, nf_ref[...], mask_ref[...], CIN)
    acc = jnp.zeros((BLK, C1), dtype=jnp.float32)
    for p in range(K):
        acc = acc + jnp.dot(w[:, p, :], ws_ref[p],
                            preferred_element_type=jnp.float32)
    x = _leaky(acc)
    x_ref[...] = x
    y1_ref[...] = _leaky(jnp.dot(x, wra_ref[...],
                                 preferred_element_type=jnp.float32))


def _tc2_body(is_last, nyr_ref, nx_ref, ny_ref, nz_ref, qx_ref, qy_ref,
              qz_ref, kp_ref, mask_ref, x_ref, wrk_ref, wrb_ref,
              wsc_ref, gprev_ref, ba_ref, wh1_ref, bh1_ref, wh2_ref,
              bh2_ref, wq_ref, bq_ref, out_ref, acc_ref):
    i = pl.program_id(0)
    infl = _influence(nx_ref[...], ny_ref[...], nz_ref[...],
                      qx_ref[...], qy_ref[...], qz_ref[...], kp_ref[...])
    w = _weighted(infl, nyr_ref[...], mask_ref[...], CB)
    yacc = jnp.zeros((BLK, CB), dtype=jnp.float32)
    for p in range(K):
        yacc = yacc + jnp.dot(w[:, p, :], wrk_ref[p],
                              preferred_element_type=jnp.float32)
    y = _leaky(yacc)
    y = jnp.dot(y, wrb_ref[...], preferred_element_type=jnp.float32)
    x = x_ref[...]
    x2 = _leaky(y + jnp.dot(x, wsc_ref[...],
                            preferred_element_type=jnp.float32))
    partial = jnp.sum(x2, axis=0, keepdims=True)

    @pl.when(i == 0)
    def _():
        acc_ref[...] = gprev_ref[...] + partial

    @pl.when(i > 0)
    def _():
        acc_ref[...] = acc_ref[...] + partial

    if not is_last:
        @pl.when(i == GRID - 1)
        def _():
            out_ref[...] = acc_ref[...]
    else:
        @pl.when(i == GRID - 1)
        def _():
            g = acc_ref[...] * (1.0 / N)
            h = jnp.concatenate([g, ba_ref[...]], axis=1)
            h = jnp.maximum(0.0, jnp.dot(h, wh1_ref[...],
                                         preferred_element_type=jnp.float32)
                            + bh1_ref[...])
            h = jnp.maximum(0.0, jnp.dot(h, wh2_ref[...],
                                         preferred_element_type=jnp.float32)
                            + bh2_ref[...])
            out_ref[...] = jnp.dot(h, wq_ref[...],
                                   preferred_element_type=jnp.float32) \
                + bq_ref[...]


def _make_sc_gather_feat_pts():
    mesh = plsc.VectorSubcoreMesh(core_axis_name="c", subcore_axis_name="s",
                                  num_cores=NC, num_subcores=NS)
    return pl.kernel(
        _sc_gather_feat_pts,
        out_type=(
            jax.ShapeDtypeStruct((NH * KN, CIN), jnp.float32),
            jax.ShapeDtypeStruct((NH * KN,), jnp.float32),
            jax.ShapeDtypeStruct((NH * KN,), jnp.float32),
            jax.ShapeDtypeStruct((NH * KN,), jnp.float32),
        ),
        mesh=mesh,
        compiler_params=pltpu.CompilerParams(use_tc_tiling_on_sc=False),
        scratch_types=[
            pltpu.VMEM((CHUNK,), jnp.int32),
            pltpu.VMEM((CHUNK,), jnp.int32),
            pltpu.VMEM((CHUNK, CIN), jnp.float32),
            pltpu.VMEM((CHUNK, CIN), jnp.float32),
            pltpu.VMEM((CHUNK,), jnp.float32),
            pltpu.VMEM((CHUNK,), jnp.float32),
            pltpu.VMEM((CHUNK,), jnp.float32),
            pltpu.VMEM((CHUNK,), jnp.float32),
            pltpu.VMEM((CHUNK,), jnp.float32),
            pltpu.VMEM((CHUNK,), jnp.float32),
            pltpu.SemaphoreType.DMA,
            pltpu.SemaphoreType.DMA,
            pltpu.SemaphoreType.DMA,
            pltpu.SemaphoreType.DMA,
        ],
    )


def _make_sc_gather_y():
    mesh = plsc.VectorSubcoreMesh(core_axis_name="c", subcore_axis_name="s",
                                  num_cores=NC, num_subcores=NS)
    return pl.kernel(
        _sc_gather_y,
        out_type=jax.ShapeDtypeStruct((NH * KN, CB), jnp.float32),
        mesh=mesh,
        compiler_params=pltpu.CompilerParams(use_tc_tiling_on_sc=False),
        scratch_types=[
            pltpu.VMEM((CHUNK,), jnp.int32),
            pltpu.VMEM((CHUNK,), jnp.int32),
            pltpu.VMEM((CHUNK, CB), jnp.float32),
            pltpu.VMEM((CHUNK, CB), jnp.float32),
            pltpu.SemaphoreType.DMA,
            pltpu.SemaphoreType.DMA,
            pltpu.SemaphoreType.DMA,
            pltpu.SemaphoreType.DMA,
        ],
    )


def _plane_specs():
    return [
        pl.BlockSpec((BLK, KN), lambda i: (i, 0)),
        pl.BlockSpec((BLK, KN), lambda i: (i, 0)),
        pl.BlockSpec((BLK, KN), lambda i: (i, 0)),
        pl.BlockSpec((BLK, 1), lambda i: (i, 0)),
        pl.BlockSpec((BLK, 1), lambda i: (i, 0)),
        pl.BlockSpec((BLK, 1), lambda i: (i, 0)),
        pl.BlockSpec((3, KP, 1), lambda i: (0, 0, 0)),
        pl.BlockSpec((SROW, SCOL), lambda i: (0, 0)),
    ]


def _make_tc1():
    return pl.pallas_call(
        _tc1_body,
        grid=(GRID,),
        in_specs=[pl.BlockSpec((BLK * KN, CIN), lambda i: (i, 0))]
        + _plane_specs()
        + [
            pl.BlockSpec((K, CIN, C1), lambda i: (0, 0, 0)),
            pl.BlockSpec((C1, CB), lambda i: (0, 0)),
        ],
        out_specs=[
            pl.BlockSpec((BLK, C1), lambda i: (i, 0)),
            pl.BlockSpec((BLK, CB), lambda i: (i, 0)),
        ],
        out_shape=[
            jax.ShapeDtypeStruct((NH, C1), jnp.float32),
            jax.ShapeDtypeStruct((NH, CB), jnp.float32),
        ],
    )


def _make_tc2(is_last):
    return pl.pallas_call(
        functools.partial(_tc2_body, is_last),
        grid=(GRID,),
        in_specs=[pl.BlockSpec((BLK * KN, CB), lambda i: (i, 0))]
        + _plane_specs()
        + [
            pl.BlockSpec((BLK, C1), lambda i: (i, 0)),
            pl.BlockSpec((K, CB, CB), lambda i: (0, 0, 0)),
            pl.BlockSpec((CB, COUT), lambda i: (0, 0)),
            pl.BlockSpec((C1, COUT), lambda i: (0, 0)),
            pl.BlockSpec((1, COUT), lambda i: (0, 0)),
            pl.BlockSpec((1, A), lambda i: (0, 0)),
            pl.BlockSpec((COUT + A, H), lambda i: (0, 0)),
            pl.BlockSpec((1, H), lambda i: (0, 0)),
            pl.BlockSpec((H, H), lambda i: (0, 0)),
            pl.BlockSpec((1, H), lambda i: (0, 0)),
            pl.BlockSpec((H, 1), lambda i: (0, 0)),
            pl.BlockSpec((1, 1), lambda i: (0, 0)),
        ],
        out_specs=pl.BlockSpec(
            (1, 1) if is_last else (1, COUT), lambda i: (0, 0)),
        out_shape=jax.ShapeDtypeStruct(
            (1, 1) if is_last else (1, COUT), jnp.float32),
        scratch_shapes=[pltpu.VMEM((1, COUT), jnp.float32)],
    )


def _block_diag_mask():
    r = jnp.arange(SROW)[:, None] // KP
    c = jnp.arange(SCOL)[None, :] // KN
    return (r == c).astype(jnp.float32)


def kernel(features, points, neighbors, batch_action, kernel_points,
           W_simple, W_ra, W_rk, W_rb, W_sc, Wh1, bh1, Wh2, bh2, Wq, bq):
    nbr = neighbors.reshape(-1).astype(jnp.int32)
    pts = points.astype(jnp.float32)
    px, py, pz = pts[:, 0], pts[:, 1], pts[:, 2]
    kp = jnp.pad(kernel_points.astype(jnp.float32).T, ((0, 0), (0, 1)),
                 constant_values=1e3).reshape(3, KP, 1)
    mask = _block_diag_mask()

    sc1 = _make_sc_gather_feat_pts()
    tc1 = _make_tc1()
    scy = _make_sc_gather_y()

    halves = []
    for h_i in range(N_SPLIT):
        nbr_h = lax.slice_in_dim(nbr, h_i * NH * KN, (h_i + 1) * NH * KN)
        nf_flat, nx_f, ny_f, nz_f = sc1(nbr_h, features, px, py, pz)
        halves.append((
            nbr_h, nf_flat,
            nx_f.reshape(NH, KN), ny_f.reshape(NH, KN), nz_f.reshape(NH, KN),
            lax.slice_in_dim(px, h_i * NH, (h_i + 1) * NH).reshape(NH, 1),
            lax.slice_in_dim(py, h_i * NH, (h_i + 1) * NH).reshape(NH, 1),
            lax.slice_in_dim(pz, h_i * NH, (h_i + 1) * NH).reshape(NH, 1),
        ))

    xs, y1s = [], []
    for (nbr_h, nf_flat, nx, nyc, nz, qx, qy, qz) in halves:
        x_h, y1_h = tc1(nf_flat, nx, nyc, nz, qx, qy, qz, kp, mask,
                        W_simple, W_ra)
        xs.append(x_h)
        y1s.append(y1_h)
    y1 = jnp.concatenate(y1s, axis=0)

    nys = [scy(h[0], y1) for h in halves]

    gprev = jnp.zeros((1, COUT), jnp.float32)
    out = None
    for h_i in range(N_SPLIT):
        (nbr_h, nf_flat, nx, nyc, nz, qx, qy, qz) = halves[h_i]
        out = _make_tc2(h_i == N_SPLIT - 1)(
            nys[h_i], nx, nyc, nz, qx, qy, qz, kp, mask, xs[h_i],
            W_rk, W_rb, W_sc, gprev, batch_action, Wh1, bh1.reshape(1, H),
            Wh2, bh2.reshape(1, H), Wq, bq.reshape(1, 1))
        gprev = out
    return out
